# baseline (speedup 1.0000x reference)
import jax
import jax.numpy as jnp
from jax import lax
from jax.experimental import pallas as pl
from jax.experimental.pallas import tpu as pltpu
from jax.experimental.pallas import tpu_sc as plsc

_B, _C, _H, _W = 2, 96, 64, 64
_K = 19
_OH, _OW = 512, 512
_ROWS = 8
_L = 16


def _sc_body(feat_hbm, cent_hbm, out_hbm, feat_v, cent_v, bias_v, am_v,
             orow_v, sem):
    cid = lax.axis_index("c")
    sid = lax.axis_index("s")
    wid = sid * 2 + cid
    p = wid // 16
    t = wid % 16
    b = t // 8
    y0 = (t % 8) * _ROWS

    feat_cp = pltpu.async_copy(feat_hbm.at[p, b, :, pl.ds(y0 * _W, _ROWS * _W)],
                               feat_v, sem)
    pltpu.sync_copy(cent_hbm.at[p], cent_v)

    def bias_body(k, _):
        acc = jnp.zeros((_L,), jnp.float32)
        for c in range(_C):
            w = cent_v[pl.ds(k * _L + c * (_K * _L), _L)]
            acc = acc + w * w
        bias_v[pl.ds(k * _L, _L)] = acc * (-0.5)
        return 0
    lax.fori_loop(0, _K, bias_body, 0)

    feat_cp.wait()

    def main_body(i, _):
        a0 = [bias_v[pl.ds(k * _L, _L)] for k in range(_K)]
        a1 = [bias_v[pl.ds(k * _L, _L)] for k in range(_K)]
        for c in range(_C):
            f0 = feat_v[c, pl.ds(i * 32, _L)]
            f1 = feat_v[c, pl.ds(i * 32 + _L, _L)]
            base = c * (_K * _L)
            for k in range(_K):
                w = cent_v[pl.ds(base + k * _L, _L)]
                a0[k] = a0[k] + f0 * w
                a1[k] = a1[k] + f1 * w
        for g, a in ((0, a0), (1, a1)):
            best = a[0]
            bi = jnp.zeros((_L,), jnp.int32)
            for k in range(1, _K):
                m = a[k] > best
                best = jnp.where(m, a[k], best)
                bi = jnp.where(m, jnp.int32(k), bi)
            am_v[pl.ds(i * 32 + g * _L, _L)] = bi
        return 0
    lax.fori_loop(0, 16, main_body, 0)

    ioc = lax.iota(jnp.int32, _L) // 8

    def expand_body(r, _):
        for j in range(32):
            idx = r * 64 + 2 * j + ioc
            vals = plsc.load_gather(am_v, [idx])
            for rr in range(8):
                orow_v[rr, pl.ds(j * _L, _L)] = vals
        pltpu.sync_copy(orow_v, out_hbm.at[p, b, pl.ds((y0 + r) * 8, 8), :])
        return 0
    lax.fori_loop(0, _ROWS, expand_body, 0)


@jax.jit
def _run(feat, cent_rep):
    mesh = plsc.VectorSubcoreMesh(core_axis_name="c", subcore_axis_name="s")
    f = pl.kernel(
        _sc_body,
        mesh=mesh,
        out_type=jax.ShapeDtypeStruct((2, _B, _OH, _OW), jnp.int32),
        scratch_types=[
            pltpu.VMEM((_C, _ROWS * _W), jnp.float32),
            pltpu.VMEM((_C * _K * _L,), jnp.float32),
            pltpu.VMEM((_K * _L,), jnp.float32),
            pltpu.VMEM((_ROWS * _W,), jnp.int32),
            pltpu.VMEM((8, _OW), jnp.int32),
            pltpu.SemaphoreType.DMA,
        ],
        compiler_params=pltpu.CompilerParams(needs_layout_passes=False),
    )
    return f(feat, cent_rep)


def kernel(feature_s2t, feature_target, label_s2t, label_target,
           centroid_s2t, centroid_target):
    feat = jnp.stack([feature_s2t, feature_target])
    feat = feat.reshape(2, _B, _C, _H * _W)
    cent = jnp.stack([centroid_target, centroid_s2t])
    cent_rep = jnp.repeat(
        cent.transpose(0, 2, 1).reshape(2, _C * _K, 1), _L, axis=2
    ).reshape(2, _C * _K * _L)
    out = _run(feat, cent_rep)
    return (out[0], out[1])


# speedup vs baseline: 3.1093x; 3.1093x over previous
import jax
import jax.numpy as jnp
from jax import lax
from jax.experimental import pallas as pl
from jax.experimental.pallas import tpu as pltpu
from jax.experimental.pallas import tpu_sc as plsc

_B, _C, _H, _W = 2, 96, 64, 64
_HW = _H * _W
_K = 19
_OH, _OW = 512, 512
_ROWS = 8
_L = 16


def _tc_body(f0_ref, f1_ref, c0_ref, c1_ref, am0_ref, am1_ref):
    for f_ref, c_ref, am_ref in ((f0_ref, c0_ref, am0_ref),
                                 (f1_ref, c1_ref, am1_ref)):
        cent = c_ref[...]
        bias = 0.5 * jnp.sum(cent * cent, axis=1, keepdims=True)
        for b in range(_B):
            dot = jnp.dot(cent, f_ref[b],
                          precision=lax.Precision.HIGHEST,
                          preferred_element_type=jnp.float32)
            s = bias - dot
            best = s[0:1, :]
            bi = jnp.zeros((1, _HW), jnp.int32)
            for k in range(1, _K):
                sk = s[k:k + 1, :]
                m = sk < best
                best = jnp.where(m, sk, best)
                bi = jnp.where(m, jnp.int32(k), bi)
            am_ref[pl.ds(b, 1), :] = bi


def _sc_body(am0_hbm, am1_hbm, out0_hbm, out1_hbm, am_v, orow_v):
    cid = lax.axis_index("c")
    sid = lax.axis_index("s")
    wid = sid * 2 + cid
    p = wid // 16
    t = wid % 16
    b = t // 8
    y0 = (t % 8) * _ROWS

    @pl.when(p == 0)
    def _():
        pltpu.sync_copy(am0_hbm.at[b, pl.ds(y0 * _W, _ROWS * _W)], am_v)

    @pl.when(p == 1)
    def _():
        pltpu.sync_copy(am1_hbm.at[b, pl.ds(y0 * _W, _ROWS * _W)], am_v)

    ioc = lax.iota(jnp.int32, _L) // 8

    def expand_body(r, _):
        for j in range(32):
            idx = r * 64 + 2 * j + ioc
            vals = plsc.load_gather(am_v, [idx])
            for rr in range(8):
                orow_v[rr, pl.ds(j * _L, _L)] = vals

        @pl.when(p == 0)
        def _():
            pltpu.sync_copy(orow_v,
                            out0_hbm.at[b, pl.ds((y0 + r) * 8, 8), :])

        @pl.when(p == 1)
        def _():
            pltpu.sync_copy(orow_v,
                            out1_hbm.at[b, pl.ds((y0 + r) * 8, 8), :])
        return 0
    lax.fori_loop(0, _ROWS, expand_body, 0)


@jax.jit
def _run(f0, f1, c0, c1):
    am0, am1 = pl.pallas_call(
        _tc_body,
        out_shape=(
            jax.ShapeDtypeStruct((_B, _HW), jnp.int32),
            jax.ShapeDtypeStruct((_B, _HW), jnp.int32),
        ),
    )(f0, f1, c0, c1)

    mesh = plsc.VectorSubcoreMesh(core_axis_name="c", subcore_axis_name="s")
    sc = pl.kernel(
        _sc_body,
        mesh=mesh,
        out_type=(
            jax.ShapeDtypeStruct((_B, _OH, _OW), jnp.int32),
            jax.ShapeDtypeStruct((_B, _OH, _OW), jnp.int32),
        ),
        scratch_types=[
            pltpu.VMEM((_ROWS * _W,), jnp.int32),
            pltpu.VMEM((8, _OW), jnp.int32),
        ],
        compiler_params=pltpu.CompilerParams(needs_layout_passes=False),
    )
    return sc(am0, am1)


def kernel(feature_s2t, feature_target, label_s2t, label_target,
           centroid_s2t, centroid_target):
    f0 = feature_s2t.reshape(_B, _C, _HW)
    f1 = feature_target.reshape(_B, _C, _HW)
    return _run(f0, f1, centroid_target, centroid_s2t)


# speedup vs baseline: 3.1480x; 1.0125x over previous
import jax
import jax.numpy as jnp
from jax import lax
from jax.experimental import pallas as pl
from jax.experimental.pallas import tpu as pltpu
from jax.experimental.pallas import tpu_sc as plsc

_B, _C, _H, _W = 2, 96, 64, 64
_HW = _H * _W
_K = 19
_OH, _OW = 512, 512
_ROWS = 8
_L = 16


def _tc_body(f0_ref, f1_ref, c0_ref, c1_ref, am0_ref, am1_ref):
    for f_ref, c_ref, am_ref in ((f0_ref, c0_ref, am0_ref),
                                 (f1_ref, c1_ref, am1_ref)):
        cent = c_ref[...]
        bias = 0.5 * jnp.sum(cent * cent, axis=1, keepdims=True)
        dot = jnp.dot(cent, f_ref[0],
                      precision=lax.Precision.HIGHEST,
                      preferred_element_type=jnp.float32)
        s = bias - dot
        best = s[0:1, :]
        bi = jnp.zeros((1, _HW), jnp.int32)
        for k in range(1, _K):
            sk = s[k:k + 1, :]
            m = sk < best
            best = jnp.where(m, sk, best)
            bi = jnp.where(m, jnp.int32(k), bi)
        am_ref[...] = bi.reshape(1, 1, _HW)


def _sc_body(am0_hbm, am1_hbm, out0_hbm, out1_hbm, am_v, orow_v):
    cid = lax.axis_index("c")
    sid = lax.axis_index("s")
    wid = sid * 2 + cid
    p = wid // 16
    t = wid % 16
    b = t // 8
    y0 = (t % 8) * _ROWS

    @pl.when(p == 0)
    def _():
        pltpu.sync_copy(am0_hbm.at[b, 0, pl.ds(y0 * _W, _ROWS * _W)], am_v)

    @pl.when(p == 1)
    def _():
        pltpu.sync_copy(am1_hbm.at[b, 0, pl.ds(y0 * _W, _ROWS * _W)], am_v)

    ioc = lax.iota(jnp.int32, _L) // 8

    def expand_body(r, _):
        for j in range(32):
            idx = r * 64 + 2 * j + ioc
            vals = plsc.load_gather(am_v, [idx])
            for rr in range(8):
                orow_v[rr, pl.ds(j * _L, _L)] = vals

        @pl.when(p == 0)
        def _():
            pltpu.sync_copy(orow_v,
                            out0_hbm.at[b, pl.ds((y0 + r) * 8, 8), :])

        @pl.when(p == 1)
        def _():
            pltpu.sync_copy(orow_v,
                            out1_hbm.at[b, pl.ds((y0 + r) * 8, 8), :])
        return 0
    lax.fori_loop(0, _ROWS, expand_body, 0)


@jax.jit
def _run(f0, f1, c0, c1):
    am0, am1 = pl.pallas_call(
        _tc_body,
        grid=(_B,),
        in_specs=[
            pl.BlockSpec((1, _C, _HW), lambda i: (i, 0, 0)),
            pl.BlockSpec((1, _C, _HW), lambda i: (i, 0, 0)),
            pl.BlockSpec((_K, _C), lambda i: (0, 0)),
            pl.BlockSpec((_K, _C), lambda i: (0, 0)),
        ],
        out_specs=(
            pl.BlockSpec((1, 1, _HW), lambda i: (i, 0, 0)),
            pl.BlockSpec((1, 1, _HW), lambda i: (i, 0, 0)),
        ),
        out_shape=(
            jax.ShapeDtypeStruct((_B, 1, _HW), jnp.int32),
            jax.ShapeDtypeStruct((_B, 1, _HW), jnp.int32),
        ),
    )(f0, f1, c0, c1)

    mesh = plsc.VectorSubcoreMesh(core_axis_name="c", subcore_axis_name="s")
    sc = pl.kernel(
        _sc_body,
        mesh=mesh,
        out_type=(
            jax.ShapeDtypeStruct((_B, _OH, _OW), jnp.int32),
            jax.ShapeDtypeStruct((_B, _OH, _OW), jnp.int32),
        ),
        scratch_types=[
            pltpu.VMEM((_ROWS * _W,), jnp.int32),
            pltpu.VMEM((8, _OW), jnp.int32),
        ],
        compiler_params=pltpu.CompilerParams(needs_layout_passes=False),
    )
    return sc(am0, am1)


def kernel(feature_s2t, feature_target, label_s2t, label_target,
           centroid_s2t, centroid_target):
    f0 = feature_s2t.reshape(_B, _C, _HW)
    f1 = feature_target.reshape(_B, _C, _HW)
    return _run(f0, f1, centroid_target, centroid_s2t)
